# baseline (speedup 1.0000x reference)
import functools

import jax
import jax.numpy as jnp
from jax import lax
from jax.experimental import pallas as pl
from jax.experimental.pallas import tpu as pltpu
from jax.experimental.pallas import tpu_sc as plsc

_H = 5
_BT = 128


def _make_detile(V, D):
    info = plsc.get_sparse_core_info()
    NC, NS, L = info.num_cores, info.num_subcores, info.num_lanes
    NW = NC * NS
    assert D == 32 and V % (2 * L) == 0
    NB = V // _BT
    K1_NBUF = 6
    n_groups = (NB // NW + K1_NBUF) // K1_NBUF + 1
    tail_w = (V - NB * _BT) and 1
    S = D + 1

    mesh = plsc.VectorSubcoreMesh(core_axis_name="c", subcore_axis_name="s")

    @functools.partial(
        pl.kernel,
        out_type=jax.ShapeDtypeStruct((V * D // _BT, _BT), jnp.float32),
        mesh=mesh,
        scratch_types=[
            [pltpu.VMEM((D, _BT), jnp.float32) for _ in range(6)],
            [pltpu.VMEM((D, _BT), jnp.float32) for _ in range(6)],
            pltpu.VMEM((_BT * S,), jnp.float32),
            pltpu.VMEM((D, _BT // 2), jnp.float32),
            pltpu.VMEM((_BT // 8, _BT), jnp.float32),
            [pltpu.SemaphoreType.DMA for _ in range(6)],
            [pltpu.SemaphoreType.DMA for _ in range(6)],
        ],
        compiler_params=pltpu.CompilerParams(
            use_tc_tiling_on_sc=True, needs_layout_passes=False
        ),
    )
    def detile_kernel(tt_hbm, out_hbm, ebuf, tbuf, s1, e64, t64, gsem, wsem):
        wid = lax.axis_index("s") * NC + lax.axis_index("c")
        iota = lax.iota(jnp.int32, L)
        iota_s = iota * S

        def col0(t):
            return pl.multiple_of((wid + NW * t) * _BT, _BT)

        def start_in(t, b):
            pltpu.async_copy(
                tt_hbm.at[:, pl.ds(col0(t), _BT)], ebuf[b], gsem[b]
            )

        def drain_in(b):
            pltpu.make_async_copy(
                tt_hbm.at[:, pl.ds(0, _BT)], ebuf[b], gsem[b]
            ).wait()

        def start_out(t, b):
            pltpu.async_copy(
                tbuf[b],
                out_hbm.at[pl.ds(pl.multiple_of(col0(t) // 4, D), D)],
                wsem[b],
            )

        def drain_out(b):
            pltpu.make_async_copy(
                tbuf[b], out_hbm.at[pl.ds(0, D)], wsem[b]
            ).wait()

        def compute(b):
            @plsc.parallel_loop(0, D, unroll=8)
            def _(d):
                for j0 in range(0, _BT, L):
                    v = ebuf[b][d, pl.ds(j0, L)]
                    plsc.store_scatter(s1, [iota_s + (j0 * S + d)], v)

            @plsc.parallel_loop(0, D, unroll=8)
            def _(w4):
                for q in range(4):
                    for half in range(D // L):
                        v = s1[pl.ds((w4 * 4 + q) * S + half * L, L)]
                        tbuf[b][w4, pl.ds(q * D + half * L, L)] = v

        for b in range(K1_NBUF):
            start_in(b, b)

        def group_body(g, carry):
            for b in range(K1_NBUF):
                t = K1_NBUF * g + b

                @pl.when(wid + NW * t < NB)
                def _():
                    drain_in(b)

                    @pl.when(g >= 1)
                    def _():
                        drain_out(b)

                    compute(b)
                    start_out(t, b)

                    @pl.when(wid + NW * (t + K1_NBUF) < NB)
                    def _():
                        start_in(t + K1_NBUF, b)

            return carry

        lax.fori_loop(0, n_groups, group_body, 0)
        for b in range(K1_NBUF):
            drain_out(b)

        @pl.when((wid == NB % NW) & (tail_w == 1))
        def _():
            TW = V - NB * _BT
            pltpu.sync_copy(tt_hbm.at[:, pl.ds(NB * _BT, TW)], e64.at[:, pl.ds(0, TW)])
            for d in range(D):
                for j0 in range(0, TW, L):
                    v = e64[d, pl.ds(j0, L)]
                    plsc.store_scatter(s1, [iota_s + (j0 * S + d)], v)
            for w4 in range(TW // 4):
                for q in range(4):
                    for half in range(D // L):
                        v = s1[pl.ds((w4 * 4 + q) * S + half * L, L)]
                        t64[w4, pl.ds(q * D + half * L, L)] = v
            pltpu.sync_copy(
                t64.at[pl.ds(0, TW // 4)],
                out_hbm.at[pl.ds(V * D // _BT - TW // 4, TW // 4)],
            )

    return detile_kernel


def _make_gather(V, D, BATCH, HIST):
    info = plsc.get_sparse_core_info()
    NC, NS, L = info.num_cores, info.num_subcores, info.num_lanes
    NW = NC * NS
    assert BATCH % (NW * _BT) == 0 and D % 8 == 0 and HIST % _H == 0
    bt_per_w = BATCH // (NW * _BT)
    b_per_w = BATCH // NW
    n_hg = HIST // _H
    n_chunks = bt_per_w * n_hg
    n_dt = D // 8
    assert n_chunks % 2 == 0 and n_chunks >= 4
    n_groups = n_chunks // 2

    mesh = plsc.VectorSubcoreMesh(core_axis_name="c", subcore_axis_name="s")

    @functools.partial(
        pl.kernel,
        out_type=jax.ShapeDtypeStruct(
            (HIST, n_dt, BATCH // _BT, 8, _BT), jnp.float32
        ),
        mesh=mesh,
        scratch_types=[
            pltpu.VMEM((HIST, b_per_w), jnp.int32),
            [pltpu.VMEM((_H * _BT, D), jnp.float32) for _ in range(2)],
            [pltpu.VMEM((_H, n_dt, 1, 8, _BT + 1), jnp.float32) for _ in range(2)],
            [pltpu.SemaphoreType.DMA for _ in range(2)],
            [pltpu.SemaphoreType.DMA for _ in range(2)],
        ],
        compiler_params=pltpu.CompilerParams(
            use_tc_tiling_on_sc=False, needs_layout_passes=False
        ),
    )
    def gather_kernel(idx_hbm, table_hbm, out_hbm, idxw, emb, obuf, gsem, wsem):
        wid = lax.axis_index("s") * NC + lax.axis_index("c")

        pltpu.sync_copy(idx_hbm.at[:, pl.ds(wid * b_per_w, b_per_w)], idxw)

        def start_gather(k, b):
            bt_loc = k // n_hg
            h0 = (k % n_hg) * _H
            for i in range(_H):
                pltpu.async_copy(
                    table_hbm.at[idxw.at[h0 + i, pl.ds(bt_loc * _BT, _BT)]],
                    emb[b].at[pl.ds(i * _BT, _BT)],
                    gsem[b],
                )

        def drain_gather(b):
            for _ in range(_H):
                pltpu.make_async_copy(
                    table_hbm.at[idxw.at[0, pl.ds(0, _BT)]],
                    emb[b].at[pl.ds(0, _BT)],
                    gsem[b],
                ).wait()

        def start_write(k, b):
            bt_g = wid * bt_per_w + k // n_hg
            h0 = (k % n_hg) * _H
            pltpu.async_copy(
                obuf[b].at[:, :, :, :, pl.ds(0, _BT)],
                out_hbm.at[pl.ds(h0, _H), :, pl.ds(bt_g, 1)],
                wsem[b],
            )

        def drain_write(b):
            pltpu.make_async_copy(
                obuf[b].at[:, :, :, :, pl.ds(0, _BT)],
                out_hbm.at[pl.ds(0, _H), :, pl.ds(0, 1)],
                wsem[b],
            ).wait()

        iota = lax.iota(jnp.int32, L)

        zero16 = iota * 0
        sub_v = lax.rem(iota, 8)
        dt_vs = [iota // 8 + 2 * half for half in range(D // L)]
        h_vs = [zero16 + h_i for h_i in range(_H)]

        def transpose(b):
            @plsc.parallel_loop(0, _BT, unroll=4)
            def _(j):
                j_v = zero16 + j
                for h_i in range(_H):
                    r = h_i * _BT + j
                    for half in range(D // L):
                        v = emb[b][r, pl.ds(half * L, L)]
                        plsc.store_scatter(
                            obuf[b],
                            [h_vs[h_i], dt_vs[half], zero16, sub_v, j_v],
                            v,
                        )

        start_gather(0, 0)
        start_gather(1, 1)

        def group_body(g, carry):
            for b in range(2):
                k = 2 * g + b
                drain_gather(b)

                @pl.when(g >= 1)
                def _():
                    drain_write(b)

                transpose(b)
                start_write(k, b)

                @pl.when(g <= n_groups - 2)
                def _():
                    start_gather(k + 2, b)

            return carry

        lax.fori_loop(0, n_groups, group_body, 0)
        drain_write(0)
        drain_write(1)

    return gather_kernel


def kernel(x, table):
    V, D = table.shape
    BATCH, HIST = x.shape
    idx2 = x.T.astype(jnp.int32)
    t_lin = _make_detile(V, D)(table.T).reshape(V, D)
    a = _make_gather(V, D, BATCH, HIST)(idx2, t_lin)
    out = a.transpose(2, 4, 0, 1, 3).reshape(BATCH, HIST, D)
    return out


# speedup vs baseline: 1.1668x; 1.1668x over previous
import functools

import jax
import jax.numpy as jnp
from jax import lax
from jax.experimental import pallas as pl
from jax.experimental.pallas import tpu as pltpu
from jax.experimental.pallas import tpu_sc as plsc

_H = 5
_BT = 128


def _make_detile(V, D):
    info = plsc.get_sparse_core_info()
    NC, NS, L = info.num_cores, info.num_subcores, info.num_lanes
    NW = NC * NS
    assert D == 32 and V % (2 * L) == 0
    NB = V // _BT
    K1_NBUF = 4
    n_groups = (NB // NW + K1_NBUF) // K1_NBUF + 1
    tail_w = (V - NB * _BT) and 1
    S = D + 1

    mesh = plsc.VectorSubcoreMesh(core_axis_name="c", subcore_axis_name="s")

    @functools.partial(
        pl.kernel,
        out_type=jax.ShapeDtypeStruct((V * D // _BT, _BT), jnp.float32),
        mesh=mesh,
        scratch_types=[
            [pltpu.VMEM((D, _BT), jnp.float32) for _ in range(4)],
            [pltpu.VMEM((D, _BT), jnp.float32) for _ in range(4)],
            pltpu.VMEM((_BT * S,), jnp.float32),
            pltpu.VMEM((D, _BT // 2), jnp.float32),
            pltpu.VMEM((_BT // 8, _BT), jnp.float32),
            [pltpu.SemaphoreType.DMA for _ in range(4)],
            [pltpu.SemaphoreType.DMA for _ in range(4)],
        ],
        compiler_params=pltpu.CompilerParams(
            use_tc_tiling_on_sc=True, needs_layout_passes=False
        ),
    )
    def detile_kernel(tt_hbm, out_hbm, ebuf, tbuf, s1, e64, t64, gsem, wsem):
        wid = lax.axis_index("s") * NC + lax.axis_index("c")
        iota = lax.iota(jnp.int32, L)
        iota_s = iota * S

        def col0(t):
            return pl.multiple_of((wid + NW * t) * _BT, _BT)

        def start_in(t, b):
            pltpu.async_copy(
                tt_hbm.at[:, pl.ds(col0(t), _BT)], ebuf[b], gsem[b]
            )

        def drain_in(b):
            pltpu.make_async_copy(
                tt_hbm.at[:, pl.ds(0, _BT)], ebuf[b], gsem[b]
            ).wait()

        def start_out(t, b):
            pltpu.async_copy(
                tbuf[b],
                out_hbm.at[pl.ds(pl.multiple_of(col0(t) // 4, D), D)],
                wsem[b],
            )

        def drain_out(b):
            pltpu.make_async_copy(
                tbuf[b], out_hbm.at[pl.ds(0, D)], wsem[b]
            ).wait()

        def compute(b):
            @plsc.parallel_loop(0, D, unroll=8)
            def _(d):
                for j0 in range(0, _BT, L):
                    v = ebuf[b][d, pl.ds(j0, L)]
                    plsc.store_scatter(s1, [iota_s + (j0 * S + d)], v)

            @plsc.parallel_loop(0, D, unroll=8)
            def _(w4):
                for q in range(4):
                    for half in range(D // L):
                        v = s1[pl.ds((w4 * 4 + q) * S + half * L, L)]
                        tbuf[b][w4, pl.ds(q * D + half * L, L)] = v

        for b in range(K1_NBUF):
            start_in(b, b)

        def group_body(g, carry):
            for b in range(K1_NBUF):
                t = K1_NBUF * g + b

                @pl.when(wid + NW * t < NB)
                def _():
                    drain_in(b)

                    @pl.when(g >= 1)
                    def _():
                        drain_out(b)

                    compute(b)
                    start_out(t, b)

                    @pl.when(wid + NW * (t + K1_NBUF) < NB)
                    def _():
                        start_in(t + K1_NBUF, b)

            return carry

        lax.fori_loop(0, n_groups, group_body, 0)
        for b in range(K1_NBUF):
            drain_out(b)

        @pl.when((wid == NB % NW) & (tail_w == 1))
        def _():
            TW = V - NB * _BT
            pltpu.sync_copy(tt_hbm.at[:, pl.ds(NB * _BT, TW)], e64.at[:, pl.ds(0, TW)])
            for d in range(D):
                for j0 in range(0, TW, L):
                    v = e64[d, pl.ds(j0, L)]
                    plsc.store_scatter(s1, [iota_s + (j0 * S + d)], v)
            for w4 in range(TW // 4):
                for q in range(4):
                    for half in range(D // L):
                        v = s1[pl.ds((w4 * 4 + q) * S + half * L, L)]
                        t64[w4, pl.ds(q * D + half * L, L)] = v
            pltpu.sync_copy(
                t64.at[pl.ds(0, TW // 4)],
                out_hbm.at[pl.ds(V * D // _BT - TW // 4, TW // 4)],
            )

    return detile_kernel


def _make_gather(V, D, BATCH, HIST):
    info = plsc.get_sparse_core_info()
    NC, NS, L = info.num_cores, info.num_subcores, info.num_lanes
    NW = NC * NS
    assert BATCH % (NW * _BT) == 0 and D % 8 == 0 and HIST % _H == 0
    bt_per_w = BATCH // (NW * _BT)
    b_per_w = BATCH // NW
    n_hg = HIST // _H
    n_chunks = bt_per_w * n_hg
    n_dt = D // 8
    assert n_chunks % 2 == 0 and n_chunks >= 4
    n_groups = n_chunks // 2

    mesh = plsc.VectorSubcoreMesh(core_axis_name="c", subcore_axis_name="s")

    @functools.partial(
        pl.kernel,
        out_type=jax.ShapeDtypeStruct(
            (HIST, n_dt, BATCH // _BT, 8, _BT), jnp.float32
        ),
        mesh=mesh,
        scratch_types=[
            pltpu.VMEM((HIST, b_per_w), jnp.int32),
            [pltpu.VMEM((_H * _BT, D), jnp.float32) for _ in range(2)],
            [pltpu.VMEM((_H, n_dt, 1, 8, _BT + 1), jnp.float32) for _ in range(2)],
            [pltpu.SemaphoreType.DMA for _ in range(2)],
            [pltpu.SemaphoreType.DMA for _ in range(2)],
        ],
        compiler_params=pltpu.CompilerParams(
            use_tc_tiling_on_sc=False, needs_layout_passes=False
        ),
    )
    def gather_kernel(idx_hbm, table_hbm, out_hbm, idxw, emb, obuf, gsem, wsem):
        wid = lax.axis_index("s") * NC + lax.axis_index("c")

        pltpu.sync_copy(idx_hbm.at[:, pl.ds(wid * b_per_w, b_per_w)], idxw)

        def start_gather(k, b):
            bt_loc = k // n_hg
            h0 = (k % n_hg) * _H
            for i in range(_H):
                pltpu.async_copy(
                    table_hbm.at[idxw.at[h0 + i, pl.ds(bt_loc * _BT, _BT)]],
                    emb[b].at[pl.ds(i * _BT, _BT)],
                    gsem[b],
                )

        def drain_gather(b):
            for _ in range(_H):
                pltpu.make_async_copy(
                    table_hbm.at[idxw.at[0, pl.ds(0, _BT)]],
                    emb[b].at[pl.ds(0, _BT)],
                    gsem[b],
                ).wait()

        def start_write(k, b):
            bt_g = wid * bt_per_w + k // n_hg
            h0 = (k % n_hg) * _H
            pltpu.async_copy(
                obuf[b].at[:, :, :, :, pl.ds(0, _BT)],
                out_hbm.at[pl.ds(h0, _H), :, pl.ds(bt_g, 1)],
                wsem[b],
            )

        def drain_write(b):
            pltpu.make_async_copy(
                obuf[b].at[:, :, :, :, pl.ds(0, _BT)],
                out_hbm.at[pl.ds(0, _H), :, pl.ds(0, 1)],
                wsem[b],
            ).wait()

        iota = lax.iota(jnp.int32, L)

        zero16 = iota * 0
        sub_v = lax.rem(iota, 8)
        dt_vs = [iota // 8 + 2 * half for half in range(D // L)]
        h_vs = [zero16 + h_i for h_i in range(_H)]

        def transpose(b):
            @plsc.parallel_loop(0, _BT, unroll=4)
            def _(j):
                j_v = zero16 + j
                for h_i in range(_H):
                    r = h_i * _BT + j
                    for half in range(D // L):
                        v = emb[b][r, pl.ds(half * L, L)]
                        plsc.store_scatter(
                            obuf[b],
                            [h_vs[h_i], dt_vs[half], zero16, sub_v, j_v],
                            v,
                        )

        start_gather(0, 0)
        start_gather(1, 1)

        def group_body(g, carry):
            for b in range(2):
                k = 2 * g + b
                drain_gather(b)

                @pl.when(g >= 1)
                def _():
                    drain_write(b)

                transpose(b)
                start_write(k, b)

                @pl.when(g <= n_groups - 2)
                def _():
                    start_gather(k + 2, b)

            return carry

        lax.fori_loop(0, n_groups, group_body, 0)
        drain_write(0)
        drain_write(1)

    return gather_kernel


def kernel(x, table):
    V, D = table.shape
    BATCH, HIST = x.shape
    idx2 = x.T.astype(jnp.int32)
    t_lin = _make_detile(V, D)(table.T).reshape(V, D)
    a = _make_gather(V, D, BATCH, HIST)(idx2, t_lin)
    out = a.transpose(2, 4, 0, 1, 3).reshape(BATCH, HIST, D)
    return out
